# baseline (speedup 1.0000x reference)
import functools

import jax
import jax.numpy as jnp
from jax import lax
from jax.experimental import pallas as pl
from jax.experimental.pallas import tpu as pltpu
from jax.experimental.pallas import tpu_sc as plsc

LANES = 16
NUM_CORES = 2
NUM_SUBCORES = 16
NW = NUM_CORES * NUM_SUBCORES

SUPER = 200
GPARTS = (80, 80, 40)
NBUF = 2


def _score_body(ew, supers, dim,
                node_hbm, edge_hbm, src_hbm, dst_hbm, out_hbm,
                idx_s, idx_d, head, tail, rel, outb, sem0, sem1, osem0, osem1):
    wpr = dim // 2
    base = (lax.axis_index("s") * NUM_CORES + lax.axis_index("c")) * ew
    sems = (sem0, sem1)
    osems = (osem0, osem1)

    pltpu.sync_copy(src_hbm.at[pl.ds(base, ew)], idx_s)
    pltpu.sync_copy(dst_hbm.at[pl.ds(base, ew)], idx_d)

    def issue(c, b):
        off = c * SUPER
        part = 0
        for g in GPARTS:
            pltpu.async_copy(node_hbm.at[idx_s.at[pl.ds(off + part, g)]],
                             head.at[b, pl.ds(part, g)], sems[b])
            pltpu.async_copy(node_hbm.at[idx_d.at[pl.ds(off + part, g)]],
                             tail.at[b, pl.ds(part, g)], sems[b])
            part += g
        pltpu.async_copy(edge_hbm.at[pl.ds(base + off, SUPER)],
                         rel.at[b], sems[b])

    def drain(b):
        for g in GPARTS:
            pltpu.make_async_copy(node_hbm.at[pl.ds(0, g)],
                                  head.at[b, pl.ds(0, g)], sems[b]).wait()
            pltpu.make_async_copy(node_hbm.at[pl.ds(0, g)],
                                  tail.at[b, pl.ds(0, g)], sems[b]).wait()
        pltpu.make_async_copy(edge_hbm.at[pl.ds(0, SUPER)],
                              rel.at[b], sems[b]).wait()

    def drain_out(b):
        pltpu.make_async_copy(out_hbm.at[pl.ds(0, SUPER)],
                              outb.at[b, pl.ds(0, SUPER)], osems[b]).wait()

    lane = lax.iota(jnp.int32, LANES)
    perms = [lane ^ sh for sh in (1, 2, 4, 8)]
    lane0 = lane == 0

    def compute(c, b):
        @plsc.parallel_loop(0, SUPER, step=1, unroll=4)
        def edge_body(e):
            acc = [None, None]
            for j in range(wpr // LANES):
                hw = plsc.bitcast(head[b, e, pl.ds(j * LANES, LANES)],
                                  jnp.bfloat16)
                tw = plsc.bitcast(tail[b, e, pl.ds(j * LANES, LANES)],
                                  jnp.bfloat16)
                ht_lo, ht_hi = plsc.unpack(hw * tw,
                                           format=plsc.PackFormat.INTERLEAVED)
                r_lo = rel[b, e, pl.ds(j * 2 * LANES, LANES)]
                r_hi = rel[b, e, pl.ds(j * 2 * LANES + LANES, LANES)]
                p0 = ht_lo * r_lo
                p1 = ht_hi * r_hi
                acc[0] = p0 if acc[0] is None else acc[0] + p0
                acc[1] = p1 if acc[1] is None else acc[1] + p1
            v = acc[0] + acc[1]
            for p_ix in perms:
                v = v + v.at[p_ix].get(mode="promise_in_bounds")
            plsc.store_compressed(outb.at[b, pl.ds(e, LANES)], v, mask=lane0)

    for b in range(NBUF):
        issue(b, b)

    def outer(i, _):
        for b in range(NBUF):
            c = i * NBUF + b
            drain(b)

            @pl.when(c >= NBUF)
            def _():
                drain_out(b)

            compute(c, b)
            pltpu.async_copy(outb.at[b, pl.ds(0, SUPER)],
                             out_hbm.at[pl.ds(base + c * SUPER, SUPER)],
                             osems[b])

            @pl.when(c + NBUF < supers)
            def _():
                issue(c + NBUF, b)
        return 0

    lax.fori_loop(0, supers // NBUF, outer, 0)
    for b in range(NBUF):
        drain_out(b)


def kernel(node_emb, edge_emb, src, dst):
    n_nodes, dim = node_emb.shape
    n_edges, _ = edge_emb.shape
    assert n_edges % (NW * SUPER) == 0 and dim % (2 * LANES) == 0
    assert (n_edges // (NW * SUPER)) % NBUF == 0
    assert sum(GPARTS) == SUPER
    ew = n_edges // NW
    supers = ew // SUPER

    mesh = plsc.VectorSubcoreMesh(core_axis_name="c", subcore_axis_name="s")
    params = pltpu.CompilerParams(needs_layout_passes=False,
                                  use_tc_tiling_on_sc=False)

    node_packed = jax.lax.bitcast_convert_type(
        node_emb.astype(jnp.bfloat16)
        .reshape(n_nodes, dim // 32, 2, 16)
        .swapaxes(2, 3)
        .reshape(n_nodes, dim // 2, 2),
        jnp.int32)

    score = pl.kernel(
        functools.partial(_score_body, ew, supers, dim),
        out_type=jax.ShapeDtypeStruct((n_edges,), jnp.float32),
        mesh=mesh,
        compiler_params=params,
        scratch_types=[
            pltpu.VMEM((ew,), jnp.int32),
            pltpu.VMEM((ew,), jnp.int32),
            pltpu.VMEM((NBUF, SUPER, dim // 2), jnp.int32),
            pltpu.VMEM((NBUF, SUPER, dim // 2), jnp.int32),
            pltpu.VMEM((NBUF, SUPER, dim), jnp.float32),
            pltpu.VMEM((NBUF, SUPER + LANES), jnp.float32),
            pltpu.SemaphoreType.DMA,
            pltpu.SemaphoreType.DMA,
            pltpu.SemaphoreType.DMA,
            pltpu.SemaphoreType.DMA,
        ],
    )
    return score(node_packed, edge_emb,
                 src.astype(jnp.int32), dst.astype(jnp.int32))
